# baseline (speedup 1.0000x reference)
import functools

import jax
import jax.numpy as jnp
from jax import lax
from jax.experimental import pallas as pl
from jax.experimental.pallas import tpu as pltpu
from jax.experimental.pallas import tpu_sc as plsc

_TB = 2048
_E = 16
_L = 16
_NW = 32
_NEG_INF = float("-inf")


def _matmul_body(w_ref, hs_ref, out_ref):
    out_ref[...] = jax.lax.dot_general(
        w_ref[...], hs_ref[...], (((1,), (1,)), ((), ())),
        preferred_element_type=jnp.float32,
    )


def _logits_T(hs, W):
    T, d = hs.shape
    return pl.pallas_call(
        _matmul_body,
        grid=(T // _TB,),
        in_specs=[
            pl.BlockSpec((_E, d), lambda i: (0, 0)),
            pl.BlockSpec((_TB, d), lambda i: (i, 0)),
        ],
        out_specs=pl.BlockSpec((_E, _TB), lambda i: (0, i)),
        out_shape=jax.ShapeDtypeStruct((_E, T), jnp.float32),
    )(W, hs)


_GATHER_DNUMS = lax.GatherDimensionNumbers(
    offset_dims=(), collapsed_slice_dims=(0,), start_index_map=(0,))


def _gather16(src, q):
    return lax.gather(
        src, q[:, None], _GATHER_DNUMS, (1,),
        mode=lax.GatherScatterMode.PROMISE_IN_BOUNDS)


def _interleave(a, b, q, even):
    return jnp.where(even, _gather16(a, q), _gather16(b, q))


def _route_body(lg_hbm, w_hbm, e_hbm, blk, wbuf, ebuf):
    cpt = lax.axis_index("s") * 2 + lax.axis_index("c")
    chunk = blk.shape[1]
    base = cpt * chunk
    pltpu.sync_copy(lg_hbm.at[:, pl.ds(base, chunk)], blk)

    lane = lax.iota(jnp.int32, _L)
    q0 = lax.shift_right_logical(lane, 1)
    q1 = q0 + 8
    even = (lane & 1) == 0

    def group(g, carry):
        t0 = g * _L
        max1 = blk[0, pl.ds(t0, _L)]
        idx1 = jnp.zeros((_L,), jnp.int32)
        max2 = jnp.full((_L,), _NEG_INF, jnp.float32)
        idx2 = jnp.zeros((_L,), jnp.int32)
        for e in range(1, _E):
            v = blk[e, pl.ds(t0, _L)]
            ev = jnp.full((_L,), e, jnp.int32)
            gt1 = v > max1
            gt2 = v > max2
            max2n = jnp.where(gt1, max1, jnp.where(gt2, v, max2))
            idx2n = jnp.where(gt1, idx1, jnp.where(gt2, ev, idx2))
            max1 = jnp.where(gt1, v, max1)
            idx1 = jnp.where(gt1, ev, idx1)
            max2, idx2 = max2n, idx2n
        t = jnp.exp(max2 - max1)
        denom = 1.0 + t
        w1 = 1.0 / denom
        w2 = t / denom
        wbuf[pl.ds(2 * t0, _L)] = _interleave(w1, w2, q0, even)
        wbuf[pl.ds(2 * t0 + _L, _L)] = _interleave(w1, w2, q1, even)
        ebuf[pl.ds(2 * t0, _L)] = _interleave(idx1, idx2, q0, even)
        ebuf[pl.ds(2 * t0 + _L, _L)] = _interleave(idx1, idx2, q1, even)
        return carry

    lax.fori_loop(0, chunk // _L, group, 0)
    pltpu.sync_copy(wbuf, w_hbm.at[cpt])
    pltpu.sync_copy(ebuf, e_hbm.at[cpt])


def _route(logits_T):
    E, T = logits_T.shape
    chunk = T // _NW
    mesh = plsc.VectorSubcoreMesh(core_axis_name="c", subcore_axis_name="s")
    fn = functools.partial(
        pl.kernel,
        mesh=mesh,
        out_type=(
            jax.ShapeDtypeStruct((_NW, 2 * chunk), jnp.float32),
            jax.ShapeDtypeStruct((_NW, 2 * chunk), jnp.int32),
        ),
        scratch_types=[
            pltpu.VMEM((E, chunk), jnp.float32),
            pltpu.VMEM((2 * chunk,), jnp.float32),
            pltpu.VMEM((2 * chunk,), jnp.int32),
        ],
    )(_route_body)
    w, e = fn(logits_T)
    return w.reshape(T, 2), e.reshape(T, 2)


@jax.jit
def kernel(hidden_states, W):
    hs = hidden_states.reshape(-1, hidden_states.shape[-1])
    lt = _logits_T(hs, W)
    top_weights, top_experts = _route(lt)
    return (top_weights, top_experts)


# speedup vs baseline: 1.6069x; 1.6069x over previous
import jax
import jax.numpy as jnp
from jax import lax
from jax.experimental import pallas as pl

_TB = 2048
_E = 16
_NEG_INF = float("-inf")


def _body(w_ref, hs_ref, wout_ref, eout_ref):
    lg = jax.lax.dot_general(
        w_ref[...], hs_ref[...], (((1,), (1,)), ((), ())),
        preferred_element_type=jnp.float32,
    )
    row = lax.broadcasted_iota(jnp.int32, lg.shape, 0)
    m1 = jnp.max(lg, axis=0, keepdims=True)
    i1 = jnp.min(jnp.where(lg == m1, row, _E), axis=0, keepdims=True)
    masked = jnp.where(row == i1, _NEG_INF, lg)
    m2 = jnp.max(masked, axis=0, keepdims=True)
    i2 = jnp.min(jnp.where(masked == m2, row, _E), axis=0, keepdims=True)
    t = jnp.exp(m2 - m1)
    denom = 1.0 + t
    w1 = 1.0 / denom
    w2 = t / denom
    packed = jnp.concatenate(
        [w1, w2, lax.bitcast_convert_type(i1, jnp.float32),
         lax.bitcast_convert_type(i2, jnp.float32)],
        axis=0,
    )
    pt = packed.T
    wout_ref[...] = pt[:, :2]
    eout_ref[...] = lax.bitcast_convert_type(pt[:, 2:4], jnp.int32)


@jax.jit
def kernel(hidden_states, W):
    hs = hidden_states.reshape(-1, hidden_states.shape[-1])
    T, d = hs.shape
    tw, te = pl.pallas_call(
        _body,
        grid=(T // _TB,),
        in_specs=[
            pl.BlockSpec((_E, d), lambda i: (0, 0)),
            pl.BlockSpec((_TB, d), lambda i: (i, 0)),
        ],
        out_specs=(
            pl.BlockSpec((_TB, 2), lambda i: (i, 0)),
            pl.BlockSpec((_TB, 2), lambda i: (i, 0)),
        ),
        out_shape=(
            jax.ShapeDtypeStruct((T, 2), jnp.float32),
            jax.ShapeDtypeStruct((T, 2), jnp.int32),
        ),
    )(W, hs)
    return (tw, te)


# speedup vs baseline: 2.1730x; 1.3523x over previous
import jax
import jax.numpy as jnp
from jax import lax
from jax.experimental import pallas as pl

_TB = 2048
_E = 16
_NEG_INF = float("-inf")


def _body(w_ref, hs_ref, wout_ref, eout_ref):
    lg = jax.lax.dot_general(
        w_ref[...], hs_ref[...], (((1,), (1,)), ((), ())),
        preferred_element_type=jnp.float32,
    )
    row = lax.broadcasted_iota(jnp.int32, lg.shape, 0)
    m1 = jnp.max(lg, axis=0, keepdims=True)
    i1 = jnp.min(jnp.where(lg == m1, row, _E), axis=0, keepdims=True)
    masked = jnp.where(row == i1, _NEG_INF, lg)
    m2 = jnp.max(masked, axis=0, keepdims=True)
    i2 = jnp.min(jnp.where(masked == m2, row, _E), axis=0, keepdims=True)
    t = jnp.exp(m2 - m1)
    denom = 1.0 + t
    wout_ref[...] = jnp.concatenate([1.0 / denom, t / denom], axis=0)
    eout_ref[...] = jnp.concatenate([i1, i2], axis=0)


@jax.jit
def kernel(hidden_states, W):
    hs = hidden_states.reshape(-1, hidden_states.shape[-1])
    T, d = hs.shape
    w2t, e2t = pl.pallas_call(
        _body,
        grid=(T // _TB,),
        in_specs=[
            pl.BlockSpec((_E, d), lambda i: (0, 0)),
            pl.BlockSpec((_TB, d), lambda i: (i, 0)),
        ],
        out_specs=(
            pl.BlockSpec((2, _TB), lambda i: (0, i)),
            pl.BlockSpec((2, _TB), lambda i: (0, i)),
        ),
        out_shape=(
            jax.ShapeDtypeStruct((2, T), jnp.float32),
            jax.ShapeDtypeStruct((2, T), jnp.int32),
        ),
    )(W, hs)
    return (w2t.T, e2t.T)
